# Initial kernel scaffold; baseline (speedup 1.0000x reference)
import jax
import jax.numpy as jnp
from jax.experimental import pallas as pl


def kernel(x, adj, W1, b1, W2, b2, Wfc, bfc):
    raise NotImplementedError("write your pallas kernel here")


# speedup vs baseline: 1.0715x; 1.0715x over previous
import jax
import jax.numpy as jnp
from jax.experimental import pallas as pl
from jax.experimental.pallas import tpu as pltpu

B, N, NFEAT, NHID, NCLASS = 8, 2048, 128, 16, 128


def _gcn_body(x_ref, adj_ref, w1_ref, b1_ref, w2_ref, b2_ref, wfc_ref,
              bfc_ref, out_ref):
    a = adj_ref[0]
    xb = x_ref[0]
    s1 = jnp.dot(xb, w1_ref[...], preferred_element_type=jnp.float32)
    h = jnp.maximum(
        jnp.dot(a, s1, preferred_element_type=jnp.float32) + b1_ref[...],
        0.0)
    s2 = jnp.dot(h, w2_ref[...], preferred_element_type=jnp.float32)
    g = jnp.dot(a, s2, preferred_element_type=jnp.float32) + b2_ref[...]
    logits = jnp.dot(wfc_ref[...], g,
                     preferred_element_type=jnp.float32) + bfc_ref[...]
    m = jnp.max(logits, axis=0, keepdims=True)
    shifted = logits - m
    lse = jnp.log(jnp.sum(jnp.exp(shifted), axis=0, keepdims=True))
    out_ref[0] = shifted - lse


def kernel(x, adj, W1, b1, W2, b2, Wfc, bfc):
    out = pl.pallas_call(
        _gcn_body,
        grid=(B,),
        in_specs=[
            pl.BlockSpec((1, N, NFEAT), lambda b: (b, 0, 0)),
            pl.BlockSpec((1, N, N), lambda b: (b, 0, 0)),
            pl.BlockSpec((NFEAT, NHID), lambda b: (0, 0)),
            pl.BlockSpec((1, NHID), lambda b: (0, 0)),
            pl.BlockSpec((NHID, 1), lambda b: (0, 0)),
            pl.BlockSpec((1, 1), lambda b: (0, 0)),
            pl.BlockSpec((NCLASS, N), lambda b: (0, 0)),
            pl.BlockSpec((NCLASS, 1), lambda b: (0, 0)),
        ],
        out_specs=pl.BlockSpec((1, NCLASS, 1), lambda b: (b, 0, 0)),
        out_shape=jax.ShapeDtypeStruct((B, NCLASS, 1), jnp.float32),
        compiler_params=pltpu.CompilerParams(
            dimension_semantics=("arbitrary",)),
    )(x, adj, W1, b1.reshape(1, NHID), W2, b2.reshape(1, 1), Wfc,
      bfc.reshape(NCLASS, 1))
    return out[:, :, 0]
